# Initial kernel scaffold; baseline (speedup 1.0000x reference)
import jax
import jax.numpy as jnp
from jax.experimental import pallas as pl


def kernel(feat, edge_index, eps, W1, b1, W2, b2):
    raise NotImplementedError("write your pallas kernel here")


# speedup vs baseline: 4.1824x; 4.1824x over previous
import functools

import jax
import jax.numpy as jnp
from jax import lax
from jax.experimental import pallas as pl
from jax.experimental.pallas import tpu as pltpu
from jax.experimental.pallas import tpu_sc as plsc

NC = 2
NS = 16
NW = NC * NS
CHUNK = 128


def _sc_aggregate(featpad, src, dst, zrow, n_pad, dw, chunks, per_w, rt):
  mesh = plsc.VectorSubcoreMesh(core_axis_name="c", subcore_axis_name="s")

  @functools.partial(
      pl.kernel,
      mesh=mesh,
      compiler_params=pltpu.CompilerParams(use_tc_tiling_on_sc=False),
      out_type=jax.ShapeDtypeStruct((NC, n_pad, dw), jnp.float32),
      scratch_types=[
          pltpu.VMEM((CHUNK,), jnp.int32),
          pltpu.VMEM((CHUNK,), jnp.int32),
          pltpu.VMEM((CHUNK, dw), jnp.float32),
          pltpu.VMEM_SHARED((n_pad, dw), jnp.float32),
          pltpu.SemaphoreType.DMA,
      ],
  )
  def sc_agg(fp_hbm, src_hbm, dst_hbm, z_hbm, out_hbm, sidx, didx, rows,
             acc, sem):
    c = lax.axis_index("c")
    s = lax.axis_index("s")
    wid = c * NS + s
    row0 = s * rt
    for j in range(rt // CHUNK):
      pltpu.sync_copy(z_hbm, acc.at[pl.ds(row0 + j * CHUNK, CHUNK)])
    plsc.subcore_barrier()

    ebase = wid * per_w

    def body(j, carry):
      base = ebase + j * CHUNK
      pltpu.sync_copy(src_hbm.at[pl.ds(base, CHUNK)], sidx)
      pltpu.async_copy(fp_hbm.at[sidx], rows, sem).wait()
      pltpu.sync_copy(dst_hbm.at[pl.ds(base, CHUNK)], didx)
      pltpu.sync_copy(rows, acc.at[didx], add=True)
      return carry

    lax.fori_loop(0, chunks, body, 0)
    plsc.subcore_barrier()
    for j in range(rt // CHUNK):
      r = row0 + j * CHUNK
      pltpu.sync_copy(acc.at[pl.ds(r, CHUNK)], out_hbm.at[c, pl.ds(r, CHUNK)])

  return sc_agg(featpad, src, dst, zrow)


def _tc_body(eps_ref, acc_ref, feat_ref, w1_ref, b1_ref, w2_ref, b2_ref,
             out_ref):
  d = feat_ref.shape[1]
  s = acc_ref[0] + acc_ref[1]
  agg_sum = s[:, :d]
  deg = jnp.maximum(s[:, d:d + 1], 1.0)
  agg = agg_sum / deg
  f = feat_ref[...]
  rst = (1.0 + eps_ref[0, 0]) * f + agg
  z = jnp.dot(rst, w1_ref[...], preferred_element_type=jnp.float32)
  z = jnp.maximum(z + b1_ref[...], 0.0)
  z = jnp.dot(z, w2_ref[...], preferred_element_type=jnp.float32)
  z = jnp.maximum(z + b2_ref[...], 0.0)
  out_ref[...] = z + f


def kernel(feat, edge_index, eps, W1, b1, W2, b2):
  n, d = feat.shape
  e = edge_index.shape[1]
  dw = d + 16

  e_pad = -(-e // (NW * CHUNK)) * (NW * CHUNK)
  per_w = e_pad // NW
  chunks = per_w // CHUNK
  rt = -(-(n + 1) // (NS * CHUNK)) * CHUNK
  n_pad = NS * rt

  src = edge_index[0].astype(jnp.int32)
  dst = edge_index[1].astype(jnp.int32)
  pad_idx = jnp.full((e_pad - e,), n, dtype=jnp.int32)
  src = jnp.concatenate([src, pad_idx])
  dst = jnp.concatenate([dst, pad_idx])

  featpad = jnp.concatenate(
      [feat, jnp.ones((n, 1), jnp.float32),
       jnp.zeros((n, dw - d - 1), jnp.float32)], axis=1)
  featpad = jnp.pad(featpad, ((0, n_pad - n), (0, 0)))
  zrow = jnp.zeros((CHUNK, dw), jnp.float32)

  acc = _sc_aggregate(featpad, src, dst, zrow, n_pad, dw, chunks, per_w, rt)

  featn = jnp.pad(feat, ((0, n_pad - n), (0, 0)))
  rows = 1024
  grid = n_pad // rows
  out = pl.pallas_call(
      _tc_body,
      grid=(grid,),
      in_specs=[
          pl.BlockSpec(memory_space=pltpu.SMEM),
          pl.BlockSpec((NC, rows, dw), lambda i: (0, i, 0)),
          pl.BlockSpec((rows, d), lambda i: (i, 0)),
          pl.BlockSpec((d, d), lambda i: (0, 0)),
          pl.BlockSpec((1, d), lambda i: (0, 0)),
          pl.BlockSpec((d, d), lambda i: (0, 0)),
          pl.BlockSpec((1, d), lambda i: (0, 0)),
      ],
      out_specs=pl.BlockSpec((rows, d), lambda i: (i, 0)),
      out_shape=jax.ShapeDtypeStruct((n_pad, d), jnp.float32),
  )(jnp.asarray(eps, jnp.float32).reshape(1, 1), acc, featn, W1,
    b1.reshape(1, d), W2, b2.reshape(1, d))
  return out[:n]


# speedup vs baseline: 4.2366x; 1.0129x over previous
import functools

import jax
import jax.numpy as jnp
from jax import lax
from jax.experimental import pallas as pl
from jax.experimental.pallas import tpu as pltpu
from jax.experimental.pallas import tpu_sc as plsc

NC = 2
NS = 16
NW = NC * NS
CHUNK = 128
NBUF = 2


def _sc_aggregate(featpad, src, dst, zrow, n_pad, dw, chunks, per_w, rt):
  mesh = plsc.VectorSubcoreMesh(core_axis_name="c", subcore_axis_name="s")

  @functools.partial(
      pl.kernel,
      mesh=mesh,
      compiler_params=pltpu.CompilerParams(use_tc_tiling_on_sc=False),
      out_type=jax.ShapeDtypeStruct((NC, n_pad, dw), jnp.float32),
      scratch_types=[
          pltpu.VMEM((NBUF, CHUNK), jnp.int32),
          pltpu.VMEM((NBUF, CHUNK), jnp.int32),
          pltpu.VMEM((NBUF, CHUNK, dw), jnp.float32),
          pltpu.VMEM_SHARED((n_pad, dw), jnp.float32),
          pltpu.SemaphoreType.DMA((NBUF,)),
      ],
  )
  def sc_agg(fp_hbm, src_hbm, dst_hbm, z_hbm, out_hbm, sidx, didx, rows,
             acc, sem):
    c = lax.axis_index("c")
    s = lax.axis_index("s")
    wid = c * NS + s
    row0 = s * rt
    for j in range(rt // CHUNK):
      pltpu.sync_copy(z_hbm, acc.at[pl.ds(row0 + j * CHUNK, CHUNK)])
    plsc.subcore_barrier()

    ebase = wid * per_w

    def gather_cp(b):
      return pltpu.make_async_copy(fp_hbm.at[sidx.at[b]], rows.at[b],
                                   sem.at[b])

    def prefetch(j, b):
      pltpu.sync_copy(src_hbm.at[pl.ds(ebase + j * CHUNK, CHUNK)],
                      sidx.at[b])
      pltpu.sync_copy(dst_hbm.at[pl.ds(ebase + j * CHUNK, CHUNK)],
                      didx.at[b])
      gather_cp(b).start()

    for b in range(NBUF):
      prefetch(b, b)

    def outer(i, carry):
      for b in range(NBUF):
        j = i * NBUF + b
        gather_cp(b).wait()
        pltpu.sync_copy(rows.at[b], acc.at[didx.at[b]], add=True)
        prefetch(j + NBUF, b)
      return carry

    lax.fori_loop(0, chunks // NBUF, outer, 0)
    for b in range(NBUF):
      gather_cp(b).wait()
    plsc.subcore_barrier()
    for j in range(rt // CHUNK):
      r = row0 + j * CHUNK
      pltpu.sync_copy(acc.at[pl.ds(r, CHUNK)], out_hbm.at[c, pl.ds(r, CHUNK)])

  return sc_agg(featpad, src, dst, zrow)


def _tc_body(eps_ref, acc_ref, feat_ref, w1_ref, b1_ref, w2_ref, b2_ref,
             out_ref):
  d = feat_ref.shape[1]
  s = acc_ref[0] + acc_ref[1]
  agg_sum = s[:, :d]
  deg = jnp.maximum(s[:, d:d + 1], 1.0)
  agg = agg_sum / deg
  f = feat_ref[...]
  rst = (1.0 + eps_ref[0, 0]) * f + agg
  z = jnp.dot(rst, w1_ref[...], preferred_element_type=jnp.float32)
  z = jnp.maximum(z + b1_ref[...], 0.0)
  z = jnp.dot(z, w2_ref[...], preferred_element_type=jnp.float32)
  z = jnp.maximum(z + b2_ref[...], 0.0)
  out_ref[...] = z + f


def kernel(feat, edge_index, eps, W1, b1, W2, b2):
  n, d = feat.shape
  e = edge_index.shape[1]
  dw = d + 16

  e_pad = -(-e // (NW * CHUNK * NBUF)) * (NW * CHUNK * NBUF)
  per_w = e_pad // NW
  chunks = per_w // CHUNK
  rt = -(-(n + 1) // (NS * CHUNK)) * CHUNK
  n_pad = NS * rt

  src = edge_index[0].astype(jnp.int32)
  dst = edge_index[1].astype(jnp.int32)
  pad_idx = jnp.full((e_pad - e + NBUF * CHUNK,), n, dtype=jnp.int32)
  src = jnp.concatenate([src, pad_idx])
  dst = jnp.concatenate([dst, pad_idx])

  featpad = jnp.concatenate(
      [feat, jnp.ones((n, 1), jnp.float32),
       jnp.zeros((n, dw - d - 1), jnp.float32)], axis=1)
  featpad = jnp.pad(featpad, ((0, n_pad - n), (0, 0)))
  zrow = jnp.zeros((CHUNK, dw), jnp.float32)

  acc = _sc_aggregate(featpad, src, dst, zrow, n_pad, dw, chunks, per_w, rt)

  featn = jnp.pad(feat, ((0, n_pad - n), (0, 0)))
  rows = 1024
  grid = n_pad // rows
  out = pl.pallas_call(
      _tc_body,
      grid=(grid,),
      in_specs=[
          pl.BlockSpec(memory_space=pltpu.SMEM),
          pl.BlockSpec((NC, rows, dw), lambda i: (0, i, 0)),
          pl.BlockSpec((rows, d), lambda i: (i, 0)),
          pl.BlockSpec((d, d), lambda i: (0, 0)),
          pl.BlockSpec((1, d), lambda i: (0, 0)),
          pl.BlockSpec((d, d), lambda i: (0, 0)),
          pl.BlockSpec((1, d), lambda i: (0, 0)),
      ],
      out_specs=pl.BlockSpec((rows, d), lambda i: (i, 0)),
      out_shape=jax.ShapeDtypeStruct((n_pad, d), jnp.float32),
  )(jnp.asarray(eps, jnp.float32).reshape(1, 1), acc, featn, W1,
    b1.reshape(1, d), W2, b2.reshape(1, d))
  return out[:n]
